# baseline (speedup 1.0000x reference)
import functools

import jax
import numpy as np
import jax.numpy as jnp
from jax import lax
from jax.experimental import pallas as pl
from jax.experimental.pallas import tpu as pltpu
from jax.experimental.pallas import tpu_sc as plsc

N = 10000
E = 160000
D = 128
L = 12

NC = 2
NS = 16
NW = NC * NS
C = 32
NCHUNKS = E // C
KMAX = -(-NCHUNKS // NW)
TSTEPS = (KMAX + 2) // 2
ND = 0
NP = 10240
RPT = NP // NS
NV = D // 16


_MM_R = 1000


def _mm1_body(h_ref, w_ref, b_ref, o_ref):
    o_ref[...] = jnp.dot(h_ref[...], w_ref[...],
                         preferred_element_type=jnp.float32) + b_ref[...]


def _mm1(h, w, b):
    return pl.pallas_call(
        _mm1_body,
        grid=(N // _MM_R,),
        in_specs=[
            pl.BlockSpec((_MM_R, D), lambda i: (i, 0)),
            pl.BlockSpec((D, D), lambda i: (0, 0)),
            pl.BlockSpec((1, D), lambda i: (0, 0)),
        ],
        out_specs=pl.BlockSpec((_MM_R, D), lambda i: (i, 0)),
        out_shape=jax.ShapeDtypeStruct((N, D), jnp.float32),
    )(h, w, b)


def _mm2_body(h_ref, w_ref, b_ref, o_ref):
    o_ref[0] = jnp.dot(h_ref[...], w_ref[0],
                       preferred_element_type=jnp.float32) + b_ref[0]


def _mm2(h, w2, b2):
    return pl.pallas_call(
        _mm2_body,
        grid=(2, N // _MM_R),
        in_specs=[
            pl.BlockSpec((_MM_R, D), lambda j, i: (i, 0)),
            pl.BlockSpec((1, D, D), lambda j, i: (j, 0, 0)),
            pl.BlockSpec((1, 1, D), lambda j, i: (j, 0, 0)),
        ],
        out_specs=pl.BlockSpec((1, _MM_R, D), lambda j, i: (j, i, 0)),
        out_shape=jax.ShapeDtypeStruct((2, N, D), jnp.float32),
    )(h, w2, b2)


def _post_body(acc_ref, den_ref, h_ref, g_ref, b_ref, o_ref):
    t = acc_ref[0] + acc_ref[1]
    den = den_ref[...] + 1e-9
    t = t / den
    mu = jnp.mean(t, axis=-1, keepdims=True)
    var = jnp.mean((t - mu) ** 2, axis=-1, keepdims=True)
    t = (t - mu) * lax.rsqrt(var + 1e-5) * g_ref[...] + b_ref[...]
    t = 0.5 * t * (1.0 + lax.erf(t * (2.0 ** -0.5)))
    o_ref[...] = h_ref[...] + t


def _post(acc, den, h, g, b):
    return pl.pallas_call(
        _post_body,
        grid=(N // _MM_R,),
        in_specs=[
            pl.BlockSpec((NC, _MM_R, D), lambda i: (0, i, 0)),
            pl.BlockSpec((_MM_R, 1), lambda i: (i, 0)),
            pl.BlockSpec((_MM_R, D), lambda i: (i, 0)),
            pl.BlockSpec((1, D), lambda i: (0, 0)),
            pl.BlockSpec((1, D), lambda i: (0, 0)),
        ],
        out_specs=pl.BlockSpec((_MM_R, D), lambda i: (i, 0)),
        out_shape=jax.ShapeDtypeStruct((N, D), jnp.float32),
    )(acc, den, h, g, b)


_mesh = plsc.VectorSubcoreMesh(core_axis_name="c", subcore_axis_name="s")

@functools.partial(
    pl.kernel,
    mesh=_mesh,
    out_type=[
        jax.ShapeDtypeStruct((NC, NP, D), jnp.float32),
        jax.ShapeDtypeStruct((NC, NP // D, D), jnp.float32),
    ],
    scratch_types=[
        pltpu.VMEM_SHARED((NP, D), jnp.float32),
        pltpu.VMEM_SHARED((NP // D, D), jnp.float32),
        pltpu.VMEM((2, 2 * C), jnp.int32),
        pltpu.VMEM((2, 2 * C), jnp.int32),
        pltpu.VMEM((2, C), jnp.int32),
        pltpu.VMEM((2, 2 * C, D), jnp.float32),
        pltpu.VMEM((2, C, D), jnp.float32),
        pltpu.VMEM((NP // D, D), jnp.float32),
        pltpu.VMEM((NP // D, D), jnp.float32),
        pltpu.VMEM((NP // D,), jnp.int32),
        pltpu.VMEM((D,), jnp.float32),
        pltpu.SemaphoreType.DMA,
        pltpu.SemaphoreType.DMA,
        pltpu.SemaphoreType.DMA,
        pltpu.SemaphoreType.DMA,
        pltpu.SemaphoreType.DMA,
        pltpu.SemaphoreType.DMA,
    ],
)
def _edge_kernel(fsfd_hbm, eidx_hbm, attn_hbm, zeros_hbm,
                 acc_out, den_out,
                 acc_sh, den_sh, idx_v, gidx_v, sidx_v, gath_v, o_v,
                 den_a, den_b, rows_v, attn_v,
                 sem_i0, sem_i1, sem_g0, sem_g1, sem_s0, sem_s1):
    c = lax.axis_index("c")
    s = lax.axis_index("s")
    wid = s * NC + c
    sem_i = (sem_i0, sem_i1)
    sem_g = (sem_g0, sem_g1)
    sem_s = (sem_s0, sem_s1)

    pltpu.sync_copy(zeros_hbm.at[pl.ds(s * RPT, RPT)],
                    acc_sh.at[pl.ds(s * RPT, RPT)])

    @pl.when(s == 0)
    def _():
        pltpu.sync_copy(zeros_hbm.at[pl.ds(0, NP // D)], den_sh)
    pltpu.sync_copy(zeros_hbm.at[pl.ds(0, NP // D)], den_a)
    pltpu.sync_copy(zeros_hbm.at[pl.ds(0, NP // D)], den_b)
    pltpu.sync_copy(attn_hbm, attn_v)

    lane = jnp.arange(16, dtype=jnp.int32)

    def _row_body(i, _):
        rows_v[pl.ds(i * 16, 16)] = lane + i * 16
        return ()
    lax.fori_loop(0, NP // D // 16, _row_body, ())

    attn_regs = [attn_v[pl.ds(cc * 16, 16)] for cc in range(NV)]
    perms = [lane ^ kk for kk in (1, 2, 4, 8)]
    zero16 = jnp.zeros((16,), jnp.float32)

    plsc.subcore_barrier()

    def chunk_base(k):
        return (k * NW + wid) * C

    def valid(k):
        return (k * NW + wid) < NCHUNKS

    def issue_idx(k, slot):
        pltpu.async_copy(eidx_hbm.at[pl.ds((k * NW + wid) * 2 * C, 2 * C)],
                         idx_v.at[slot], sem_i[slot])

    def wait_idx(slot):
        pltpu.make_async_copy(eidx_hbm.at[pl.ds(0, 2 * C)], idx_v.at[slot],
                              sem_i[slot]).wait()

    def build_gidx(slot):
        nmax = jnp.full((16,), N - 1, jnp.int32)
        for gi in range(C // 16):
            sv = jnp.minimum(idx_v[slot, pl.ds(gi * 16, 16)], nmax)
            gidx_v[slot, pl.ds(gi * 16, 16)] = sv
            dv = jnp.minimum(idx_v[slot, pl.ds(C + gi * 16, 16)], nmax)
            sidx_v[slot, pl.ds(gi * 16, 16)] = dv
            gidx_v[slot, pl.ds(C + gi * 16, 16)] = dv + N

    def issue_gather(slot):
        pltpu.async_copy(fsfd_hbm.at[gidx_v.at[slot]], gath_v.at[slot],
                         sem_g[slot])

    def wait_gather(slot):
        pltpu.make_async_copy(fsfd_hbm.at[gidx_v.at[slot]], gath_v.at[slot],
                              sem_g[slot]).wait()

    def issue_scatter(slot):
        pltpu.async_copy(o_v.at[slot], acc_sh.at[sidx_v.at[slot]],
                         sem_s[slot], add=True)

    def wait_scatter(slot):
        pltpu.make_async_copy(o_v.at[slot], acc_sh.at[sidx_v.at[slot]],
                              sem_s[slot]).wait()

    def compute(slot):
        def group_body(gi, _):
            dv16 = sidx_v[slot, pl.ds(gi * 16, 16)]
            for j in range(16):
                e = gi * 16 + j
                d0 = dv16[j]
                drow = lax.shift_right_logical(d0, 7)
                dlow = d0 & 127
                colg = dlow & 112
                lpos = dlow & 15
                ms = []
                for cc in range(NV):
                    v = (gath_v[slot, e, pl.ds(cc * 16, 16)]
                         + gath_v[slot, C + e, pl.ds(cc * 16, 16)])
                    v = jnp.maximum(v, v * 0.2)
                    ms.append(v * attn_regs[cc])
                sacc = (((ms[0] + ms[1]) + (ms[2] + ms[3]))
                        + ((ms[4] + ms[5]) + (ms[6] + ms[7])))
                for pm in perms:
                    sacc = sacc + sacc.at[pm].get(mode="promise_in_bounds")
                pvec = jnp.exp(sacc)
                for cc in range(NV):
                    o_v[slot, e, pl.ds(cc * 16, 16)] = (
                        gath_v[slot, e, pl.ds(cc * 16, 16)] * pvec)
                den_l = den_a if j % 2 == 0 else den_b
                vv = den_l[drow, pl.ds(colg, 16)]
                den_l[drow, pl.ds(colg, 16)] = vv + jnp.where(
                    lane == jnp.full((16,), lpos, jnp.int32), pvec, zero16)
            return ()
        lax.fori_loop(0, C // 16, group_body, ())

    pltpu.sync_copy(eidx_hbm.at[pl.ds(wid * 2 * C, 2 * C)], idx_v.at[0])
    build_gidx(0)
    issue_gather(0)
    issue_idx(1, 1)

    def step_body(t, _):
        for b in range(2):
            k = 2 * t + b

            @pl.when(valid(k))
            def _():
                wait_gather(b)

            @pl.when((k >= 1) & valid(k - 1))
            def _():
                wait_scatter(1 - b)

            @pl.when(valid(k + 1))
            def _():
                wait_idx(1 - b)
                build_gidx(1 - b)
                issue_gather(1 - b)

            @pl.when(valid(k))
            def _():
                compute(b)

            @pl.when(valid(k + 2))
            def _():
                issue_idx(k + 2, b)

            @pl.when(valid(k))
            def _():
                issue_scatter(b)
        return ()

    lax.fori_loop(0, TSTEPS, step_body, ())

    kf = 2 * TSTEPS - 1

    @pl.when(valid(kf))
    def _():
        wait_scatter(kf & 1)

    pltpu.sync_copy(den_a, den_sh.at[rows_v], add=True)
    pltpu.sync_copy(den_b, den_sh.at[rows_v], add=True)

    plsc.subcore_barrier()

    pltpu.sync_copy(acc_sh.at[pl.ds(s * RPT, RPT)],
                    acc_out.at[c].at[pl.ds(s * RPT, RPT)])

    @pl.when(s == 0)
    def _():
        pltpu.sync_copy(den_sh, den_out.at[c])


def kernel(x, edge_index, W_in, b_in, W_src, b_src, W_dst, b_dst, attn,
           ln_g, ln_b, W_out, b_out):
    ei = edge_index.astype(jnp.int32)
    eidx = jnp.stack([ei[0].reshape(NCHUNKS, C), ei[1].reshape(NCHUNKS, C)],
                     axis=1).reshape(NCHUNKS * 2 * C)
    zeros = jnp.zeros((NP, D), jnp.float32)

    h = _mm1(x, W_in, b_in.reshape(1, D))
    for i in range(L):
        w2 = jnp.stack([W_src[i], W_dst[i]])
        b2 = jnp.stack([b_src[i], b_dst[i]]).reshape(2, 1, D)
        fsfd = _mm2(h, w2, b2).reshape(2 * N, D)
        acc, den = _edge_kernel(fsfd, eidx, attn[i], zeros)
        den1 = (den[0] + den[1]).reshape(NP)[:N].reshape(N, 1)
        h = _post(acc, den1, h, ln_g[i].reshape(1, D), ln_b[i].reshape(1, D))
    return _mm1(h, W_out, b_out.reshape(1, D))


# speedup vs baseline: 2.0102x; 2.0102x over previous
import functools

import jax
import numpy as np
import jax.numpy as jnp
from jax import lax
from jax.experimental import pallas as pl
from jax.experimental.pallas import tpu as pltpu
from jax.experimental.pallas import tpu_sc as plsc

N = 10000
E = 160000
D = 128
L = 12

NC = 2
NS = 16
NW = NC * NS
C = 32
NCHUNKS = E // C
KMAX = -(-NCHUNKS // NW)
TSTEPS = (KMAX + 2) // 2
ND = 0
NP = 10240
RPT = NP // NS
NV = D // 16


_MM_R = 1000


def _mm1_body(h_ref, w_ref, b_ref, o_ref):
    o_ref[...] = jnp.dot(h_ref[...], w_ref[...],
                         preferred_element_type=jnp.float32) + b_ref[...]


def _mm1(h, w, b):
    return pl.pallas_call(
        _mm1_body,
        grid=(N // _MM_R,),
        in_specs=[
            pl.BlockSpec((_MM_R, D), lambda i: (i, 0)),
            pl.BlockSpec((D, D), lambda i: (0, 0)),
            pl.BlockSpec((1, D), lambda i: (0, 0)),
        ],
        out_specs=pl.BlockSpec((_MM_R, D), lambda i: (i, 0)),
        out_shape=jax.ShapeDtypeStruct((N, D), jnp.float32),
    )(h, w, b)


def _mm2_body(h_ref, w_ref, b_ref, o_ref):
    o_ref[0] = jnp.dot(h_ref[...], w_ref[0],
                       preferred_element_type=jnp.float32) + b_ref[0]


def _mm2(h, w2, b2):
    return pl.pallas_call(
        _mm2_body,
        grid=(2, N // _MM_R),
        in_specs=[
            pl.BlockSpec((_MM_R, D), lambda j, i: (i, 0)),
            pl.BlockSpec((1, D, D), lambda j, i: (j, 0, 0)),
            pl.BlockSpec((1, 1, D), lambda j, i: (j, 0, 0)),
        ],
        out_specs=pl.BlockSpec((1, _MM_R, D), lambda j, i: (j, i, 0)),
        out_shape=jax.ShapeDtypeStruct((2, N, D), jnp.float32),
    )(h, w2, b2)


def _post_body(acc_ref, den_ref, h_ref, g_ref, b_ref, o_ref):
    t = acc_ref[0] + acc_ref[1]
    den = den_ref[...] + 1e-9
    t = t / den
    mu = jnp.mean(t, axis=-1, keepdims=True)
    var = jnp.mean((t - mu) ** 2, axis=-1, keepdims=True)
    t = (t - mu) * lax.rsqrt(var + 1e-5) * g_ref[...] + b_ref[...]
    t = 0.5 * t * (1.0 + lax.erf(t * (2.0 ** -0.5)))
    o_ref[...] = h_ref[...] + t


def _post(acc, den, h, g, b):
    return pl.pallas_call(
        _post_body,
        grid=(N // _MM_R,),
        in_specs=[
            pl.BlockSpec((NC, _MM_R, D), lambda i: (0, i, 0)),
            pl.BlockSpec((_MM_R, 1), lambda i: (i, 0)),
            pl.BlockSpec((_MM_R, D), lambda i: (i, 0)),
            pl.BlockSpec((1, D), lambda i: (0, 0)),
            pl.BlockSpec((1, D), lambda i: (0, 0)),
        ],
        out_specs=pl.BlockSpec((_MM_R, D), lambda i: (i, 0)),
        out_shape=jax.ShapeDtypeStruct((N, D), jnp.float32),
    )(acc, den, h, g, b)


_mesh = plsc.VectorSubcoreMesh(core_axis_name="c", subcore_axis_name="s")

@functools.partial(
    pl.kernel,
    mesh=_mesh,
    out_type=[
        jax.ShapeDtypeStruct((NC, NP, D), jnp.float32),
        jax.ShapeDtypeStruct((NC, NP // D, D), jnp.float32),
    ],
    scratch_types=[
        pltpu.VMEM_SHARED((NP, D), jnp.float32),
        pltpu.VMEM_SHARED((NP // D, D), jnp.float32),
        pltpu.VMEM((2, 2 * C), jnp.int32),
        pltpu.VMEM((2, 2 * C), jnp.int32),
        pltpu.VMEM((2, C), jnp.int32),
        pltpu.VMEM((2, 2 * C, D), jnp.float32),
        pltpu.VMEM((2, C, D), jnp.float32),
        pltpu.VMEM((2, C, D), jnp.float32),
        pltpu.VMEM((2, C), jnp.int32),
        pltpu.VMEM((D,), jnp.float32),
        pltpu.SemaphoreType.DMA,
        pltpu.SemaphoreType.DMA,
        pltpu.SemaphoreType.DMA,
        pltpu.SemaphoreType.DMA,
        pltpu.SemaphoreType.DMA,
        pltpu.SemaphoreType.DMA,
        pltpu.SemaphoreType.DMA,
        pltpu.SemaphoreType.DMA,
    ],
)
def _edge_kernel(fsfd_hbm, eidx_hbm, attn_hbm, zeros_hbm,
                 acc_out, den_out,
                 acc_sh, den_sh, idx_v, gidx_v, sidx_v, gath_v, o_v,
                 o2_v, dhi_v, attn_v,
                 sem_i0, sem_i1, sem_g0, sem_g1, sem_s0, sem_s1,
                 sem_d0, sem_d1):
    c = lax.axis_index("c")
    s = lax.axis_index("s")
    wid = s * NC + c
    sem_i = (sem_i0, sem_i1)
    sem_g = (sem_g0, sem_g1)
    sem_s = (sem_s0, sem_s1)
    sem_d = (sem_d0, sem_d1)

    pltpu.sync_copy(zeros_hbm.at[pl.ds(s * RPT, RPT)],
                    acc_sh.at[pl.ds(s * RPT, RPT)])

    @pl.when(s == 0)
    def _():
        pltpu.sync_copy(zeros_hbm.at[pl.ds(0, NP // D)], den_sh)

    pltpu.sync_copy(attn_hbm, attn_v)

    lane = jnp.arange(16, dtype=jnp.int32)
    attn_regs = [attn_v[pl.ds(cc * 16, 16)] for cc in range(NV)]
    perms = [lane ^ kk for kk in (1, 2, 4, 8)]
    lanes_cc = [lane + 16 * cc for cc in range(NV)]
    zero16 = jnp.zeros((16,), jnp.float32)

    plsc.subcore_barrier()

    def chunk_base(k):
        return (k * NW + wid) * C

    def valid(k):
        return (k * NW + wid) < NCHUNKS

    def issue_idx(k, slot):
        pltpu.async_copy(eidx_hbm.at[pl.ds((k * NW + wid) * 2 * C, 2 * C)],
                         idx_v.at[slot], sem_i[slot])

    def wait_idx(slot):
        pltpu.make_async_copy(eidx_hbm.at[pl.ds(0, 2 * C)], idx_v.at[slot],
                              sem_i[slot]).wait()

    def build_gidx(slot):
        nmax = jnp.full((16,), N - 1, jnp.int32)
        for gi in range(C // 16):
            sv = jnp.minimum(idx_v[slot, pl.ds(gi * 16, 16)], nmax)
            gidx_v[slot, pl.ds(gi * 16, 16)] = sv
            dv = jnp.minimum(idx_v[slot, pl.ds(C + gi * 16, 16)], nmax)
            sidx_v[slot, pl.ds(gi * 16, 16)] = dv
            dhi_v[slot, pl.ds(gi * 16, 16)] = lax.shift_right_logical(dv, 7)
            gidx_v[slot, pl.ds(C + gi * 16, 16)] = dv + N

    def issue_gather(slot):
        pltpu.async_copy(fsfd_hbm.at[gidx_v.at[slot]], gath_v.at[slot],
                         sem_g[slot])

    def wait_gather(slot):
        pltpu.make_async_copy(fsfd_hbm.at[gidx_v.at[slot]], gath_v.at[slot],
                              sem_g[slot]).wait()

    def issue_scatter(slot):
        pltpu.async_copy(o_v.at[slot], acc_sh.at[sidx_v.at[slot]],
                         sem_s[slot], add=True)
        pltpu.async_copy(o2_v.at[slot], den_sh.at[dhi_v.at[slot]],
                         sem_d[slot], add=True)

    def wait_scatter(slot):
        pltpu.make_async_copy(o_v.at[slot], acc_sh.at[sidx_v.at[slot]],
                              sem_s[slot]).wait()
        pltpu.make_async_copy(o2_v.at[slot], den_sh.at[dhi_v.at[slot]],
                              sem_d[slot]).wait()

    def compute(slot):
        def group_body(gi, _):
            dv16 = sidx_v[slot, pl.ds(gi * 16, 16)]
            for j in range(16):
                e = gi * 16 + j
                dsplat = dv16.at[jnp.full((16,), j, jnp.int32)].get(
                    mode="promise_in_bounds")
                dlow = dsplat & 127
                ms = []
                fs_regs = []
                for cc in range(NV):
                    fs_cc = gath_v[slot, e, pl.ds(cc * 16, 16)]
                    fs_regs.append(fs_cc)
                    v = fs_cc + gath_v[slot, C + e, pl.ds(cc * 16, 16)]
                    v = jnp.maximum(v, v * 0.2)
                    ms.append(v * attn_regs[cc])
                sacc = (((ms[0] + ms[1]) + (ms[2] + ms[3]))
                        + ((ms[4] + ms[5]) + (ms[6] + ms[7])))
                for pm in perms:
                    sacc = sacc + sacc.at[pm].get(mode="promise_in_bounds")
                pvec = jnp.exp(sacc)
                for cc in range(NV):
                    o_v[slot, e, pl.ds(cc * 16, 16)] = fs_regs[cc] * pvec
                    o2_v[slot, e, pl.ds(cc * 16, 16)] = jnp.where(
                        lanes_cc[cc] == dlow, pvec, zero16)
            return ()
        lax.fori_loop(0, C // 16, group_body, ())

    pltpu.sync_copy(eidx_hbm.at[pl.ds(wid * 2 * C, 2 * C)], idx_v.at[0])
    build_gidx(0)
    issue_gather(0)
    issue_idx(1, 1)

    def step_body(t, _):
        for b in range(2):
            k = 2 * t + b

            @pl.when(valid(k))
            def _():
                wait_gather(b)

            @pl.when((k >= 1) & valid(k - 1))
            def _():
                wait_scatter(1 - b)

            @pl.when(valid(k + 1))
            def _():
                wait_idx(1 - b)
                build_gidx(1 - b)
                issue_gather(1 - b)

            @pl.when(valid(k))
            def _():
                compute(b)

            @pl.when(valid(k + 2))
            def _():
                issue_idx(k + 2, b)

            @pl.when(valid(k))
            def _():
                issue_scatter(b)
        return ()

    lax.fori_loop(0, TSTEPS, step_body, ())

    kf = 2 * TSTEPS - 1

    @pl.when(valid(kf))
    def _():
        wait_scatter(kf & 1)

    plsc.subcore_barrier()

    pltpu.sync_copy(acc_sh.at[pl.ds(s * RPT, RPT)],
                    acc_out.at[c].at[pl.ds(s * RPT, RPT)])

    @pl.when(s == 0)
    def _():
        pltpu.sync_copy(den_sh, den_out.at[c])


def kernel(x, edge_index, W_in, b_in, W_src, b_src, W_dst, b_dst, attn,
           ln_g, ln_b, W_out, b_out):
    ei = edge_index.astype(jnp.int32)
    eidx = jnp.stack([ei[0].reshape(NCHUNKS, C), ei[1].reshape(NCHUNKS, C)],
                     axis=1).reshape(NCHUNKS * 2 * C)
    zeros = jnp.zeros((NP, D), jnp.float32)

    h = _mm1(x, W_in, b_in.reshape(1, D))
    for i in range(L):
        w2 = jnp.stack([W_src[i], W_dst[i]])
        b2 = jnp.stack([b_src[i], b_dst[i]]).reshape(2, 1, D)
        fsfd = _mm2(h, w2, b2).reshape(2 * N, D)
        acc, den = _edge_kernel(fsfd, eidx, attn[i], zeros)
        den1 = (den[0] + den[1]).reshape(NP)[:N].reshape(N, 1)
        h = _post(acc, den1, h, ln_g[i].reshape(1, D), ln_b[i].reshape(1, D))
    return _mm1(h, W_out, b_out.reshape(1, D))


# speedup vs baseline: 2.0104x; 1.0001x over previous
import functools

import jax
import numpy as np
import jax.numpy as jnp
from jax import lax
from jax.experimental import pallas as pl
from jax.experimental.pallas import tpu as pltpu
from jax.experimental.pallas import tpu_sc as plsc

N = 10000
E = 160000
D = 128
L = 12

NC = 2
NS = 16
NW = NC * NS
C = 32
NCHUNKS = E // C
KMAX = -(-NCHUNKS // NW)
TSTEPS = (KMAX + 2) // 2
NP = 10240
RPT = NP // NS
NV = D // 16


_MM_R = 1000


def _mm1_body(h_ref, w_ref, b_ref, o_ref):
    o_ref[...] = jnp.dot(h_ref[...], w_ref[...],
                         preferred_element_type=jnp.float32) + b_ref[...]


def _mm1(h, w, b):
    return pl.pallas_call(
        _mm1_body,
        grid=(N // _MM_R,),
        in_specs=[
            pl.BlockSpec((_MM_R, D), lambda i: (i, 0)),
            pl.BlockSpec((D, D), lambda i: (0, 0)),
            pl.BlockSpec((1, D), lambda i: (0, 0)),
        ],
        out_specs=pl.BlockSpec((_MM_R, D), lambda i: (i, 0)),
        out_shape=jax.ShapeDtypeStruct((N, D), jnp.float32),
    )(h, w, b)


def _mm2_body(h_ref, w_ref, b_ref, o_ref):
    o_ref[0] = jnp.dot(h_ref[...], w_ref[0],
                       preferred_element_type=jnp.float32) + b_ref[0]


def _mm2(h, w2, b2):
    return pl.pallas_call(
        _mm2_body,
        grid=(2, N // _MM_R),
        in_specs=[
            pl.BlockSpec((_MM_R, D), lambda j, i: (i, 0)),
            pl.BlockSpec((1, D, D), lambda j, i: (j, 0, 0)),
            pl.BlockSpec((1, 1, D), lambda j, i: (j, 0, 0)),
        ],
        out_specs=pl.BlockSpec((1, _MM_R, D), lambda j, i: (j, i, 0)),
        out_shape=jax.ShapeDtypeStruct((2, N, D), jnp.float32),
    )(h, w2, b2)


def _post_body(acc_ref, den_ref, h_ref, g_ref, b_ref, o_ref):
    t = acc_ref[0] + acc_ref[1]
    den = den_ref[...] + 1e-9
    t = t / den
    mu = jnp.mean(t, axis=-1, keepdims=True)
    var = jnp.mean((t - mu) ** 2, axis=-1, keepdims=True)
    t = (t - mu) * lax.rsqrt(var + 1e-5) * g_ref[...] + b_ref[...]
    t = 0.5 * t * (1.0 + lax.erf(t * (2.0 ** -0.5)))
    o_ref[...] = h_ref[...] + t


def _post(acc, den, h, g, b):
    return pl.pallas_call(
        _post_body,
        grid=(N // _MM_R,),
        in_specs=[
            pl.BlockSpec((NC, _MM_R, D), lambda i: (0, i, 0)),
            pl.BlockSpec((_MM_R, 1), lambda i: (i, 0)),
            pl.BlockSpec((_MM_R, D), lambda i: (i, 0)),
            pl.BlockSpec((1, D), lambda i: (0, 0)),
            pl.BlockSpec((1, D), lambda i: (0, 0)),
        ],
        out_specs=pl.BlockSpec((_MM_R, D), lambda i: (i, 0)),
        out_shape=jax.ShapeDtypeStruct((N, D), jnp.float32),
    )(acc, den, h, g, b)


_mesh = plsc.VectorSubcoreMesh(core_axis_name="c", subcore_axis_name="s")

@functools.partial(
    pl.kernel,
    mesh=_mesh,
    out_type=[
        jax.ShapeDtypeStruct((NC, NP, D), jnp.float32),
        jax.ShapeDtypeStruct((NC, NP // D, D), jnp.float32),
    ],
    scratch_types=[
        pltpu.VMEM_SHARED((NP, D), jnp.float32),
        pltpu.VMEM_SHARED((NP // D, D), jnp.float32),
        pltpu.VMEM((2, 2 * C), jnp.int32),
        pltpu.VMEM((2, 2 * C), jnp.int32),
        pltpu.VMEM((2, C), jnp.int32),
        pltpu.VMEM((2, 2 * C, D), jnp.float32),
        pltpu.VMEM((2, C, D), jnp.float32),
        pltpu.VMEM((2, C, D), jnp.float32),
        pltpu.VMEM((2, C), jnp.int32),
        pltpu.VMEM((D,), jnp.float32),
        pltpu.SemaphoreType.DMA,
        pltpu.SemaphoreType.DMA,
        pltpu.SemaphoreType.DMA,
        pltpu.SemaphoreType.DMA,
        pltpu.SemaphoreType.DMA,
        pltpu.SemaphoreType.DMA,
        pltpu.SemaphoreType.DMA,
        pltpu.SemaphoreType.DMA,
    ],
)
def _edge_kernel(fsfd_hbm, eidx_hbm, attn_hbm, zeros_hbm,
                 acc_out, den_out,
                 acc_sh, den_sh, idx_v, gidx_v, sidx_v, gath_v, o_v,
                 o2_v, dhi_v, attn_v,
                 sem_i0, sem_i1, sem_g0, sem_g1, sem_s0, sem_s1,
                 sem_d0, sem_d1):
    c = lax.axis_index("c")
    s = lax.axis_index("s")
    wid = s * NC + c
    sem_i = (sem_i0, sem_i1)
    sem_g = (sem_g0, sem_g1)
    sem_s = (sem_s0, sem_s1)
    sem_d = (sem_d0, sem_d1)

    pltpu.sync_copy(zeros_hbm.at[pl.ds(s * RPT, RPT)],
                    acc_sh.at[pl.ds(s * RPT, RPT)])

    @pl.when(s == 0)
    def _():
        pltpu.sync_copy(zeros_hbm.at[pl.ds(0, NP // D)], den_sh)

    pltpu.sync_copy(attn_hbm, attn_v)

    lane = jnp.arange(16, dtype=jnp.int32)
    attn_regs = [attn_v[pl.ds(cc * 16, 16)] for cc in range(NV)]
    perms = [lane ^ kk for kk in (1, 2, 4, 8)]
    lanes_cc = [lane + 16 * cc for cc in range(NV)]
    zero16 = jnp.zeros((16,), jnp.float32)

    plsc.subcore_barrier()

    def valid(k):
        return (k * NW + wid) < NCHUNKS

    def issue_idx(k, slot):
        pltpu.async_copy(eidx_hbm.at[pl.ds((k * NW + wid) * 2 * C, 2 * C)],
                         idx_v.at[slot], sem_i[slot])

    def wait_idx(slot):
        pltpu.make_async_copy(eidx_hbm.at[pl.ds(0, 2 * C)], idx_v.at[slot],
                              sem_i[slot]).wait()

    def build_gidx(slot):
        nmax = jnp.full((16,), N - 1, jnp.int32)
        for gi in range(C // 16):
            sv = jnp.minimum(idx_v[slot, pl.ds(gi * 16, 16)], nmax)
            gidx_v[slot, pl.ds(gi * 16, 16)] = sv
            dv = jnp.minimum(idx_v[slot, pl.ds(C + gi * 16, 16)], nmax)
            sidx_v[slot, pl.ds(gi * 16, 16)] = dv
            dhi_v[slot, pl.ds(gi * 16, 16)] = lax.shift_right_logical(dv, 7)
            gidx_v[slot, pl.ds(C + gi * 16, 16)] = dv + N

    def issue_gather(slot):
        pltpu.async_copy(fsfd_hbm.at[gidx_v.at[slot]], gath_v.at[slot],
                         sem_g[slot])

    def wait_gather(slot):
        pltpu.make_async_copy(fsfd_hbm.at[gidx_v.at[slot]], gath_v.at[slot],
                              sem_g[slot]).wait()

    def issue_scatter(slot):
        pltpu.async_copy(o_v.at[slot], acc_sh.at[sidx_v.at[slot]],
                         sem_s[slot], add=True)
        pltpu.async_copy(o2_v.at[slot], den_sh.at[dhi_v.at[slot]],
                         sem_d[slot], add=True)

    def wait_scatter(slot):
        pltpu.make_async_copy(o_v.at[slot], acc_sh.at[sidx_v.at[slot]],
                              sem_s[slot]).wait()
        pltpu.make_async_copy(o2_v.at[slot], den_sh.at[dhi_v.at[slot]],
                              sem_d[slot]).wait()

    def compute(slot):
        def group_body(gi, _):
            dv16 = sidx_v[slot, pl.ds(gi * 16, 16)]
            for j in range(16):
                e = gi * 16 + j
                dsplat = dv16.at[jnp.full((16,), j, jnp.int32)].get(
                    mode="promise_in_bounds")
                dlow = dsplat & 127
                ms = []
                fs_regs = []
                for cc in range(NV):
                    fs_cc = gath_v[slot, e, pl.ds(cc * 16, 16)]
                    fs_regs.append(fs_cc)
                    v = fs_cc + gath_v[slot, C + e, pl.ds(cc * 16, 16)]
                    v = jnp.maximum(v, v * 0.2)
                    ms.append(v * attn_regs[cc])
                sacc = (((ms[0] + ms[1]) + (ms[2] + ms[3]))
                        + ((ms[4] + ms[5]) + (ms[6] + ms[7])))
                for pm in perms:
                    sacc = sacc + sacc.at[pm].get(mode="promise_in_bounds")
                pvec = jnp.exp(sacc)
                for cc in range(NV):
                    o_v[slot, e, pl.ds(cc * 16, 16)] = fs_regs[cc] * pvec
                    o2_v[slot, e, pl.ds(cc * 16, 16)] = jnp.where(
                        lanes_cc[cc] == dlow, pvec, zero16)
            return ()
        lax.fori_loop(0, C // 16, group_body, ())

    pltpu.sync_copy(eidx_hbm.at[pl.ds(wid * 2 * C, 2 * C)], idx_v.at[0])
    build_gidx(0)
    issue_gather(0)
    issue_idx(1, 1)

    def step_body(t, _):
        for b in range(2):
            k = 2 * t + b

            @pl.when(valid(k))
            def _():
                wait_gather(b)

            @pl.when((k >= 1) & valid(k - 1))
            def _():
                wait_scatter(1 - b)

            @pl.when(valid(k + 1))
            def _():
                wait_idx(1 - b)
                build_gidx(1 - b)
                issue_gather(1 - b)

            @pl.when(valid(k))
            def _():
                compute(b)

            @pl.when(valid(k + 2))
            def _():
                issue_idx(k + 2, b)

            @pl.when(valid(k))
            def _():
                issue_scatter(b)
        return ()

    lax.fori_loop(0, TSTEPS, step_body, ())

    kf = 2 * TSTEPS - 1

    @pl.when(valid(kf))
    def _():
        wait_scatter(kf & 1)

    plsc.subcore_barrier()

    pltpu.sync_copy(acc_sh.at[pl.ds(s * RPT, RPT)],
                    acc_out.at[c].at[pl.ds(s * RPT, RPT)])

    @pl.when(s == 0)
    def _():
        pltpu.sync_copy(den_sh, den_out.at[c])


def kernel(x, edge_index, W_in, b_in, W_src, b_src, W_dst, b_dst, attn,
           ln_g, ln_b, W_out, b_out):
    ei = edge_index.astype(jnp.int32)
    eidx = jnp.stack([ei[0].reshape(NCHUNKS, C), ei[1].reshape(NCHUNKS, C)],
                     axis=1).reshape(NCHUNKS * 2 * C)
    zeros = jnp.zeros((NP, D), jnp.float32)

    h = _mm1(x, W_in, b_in.reshape(1, D))
    for i in range(L):
        w2 = jnp.stack([W_src[i], W_dst[i]])
        b2 = jnp.stack([b_src[i], b_dst[i]]).reshape(2, 1, D)
        fsfd = _mm2(h, w2, b2).reshape(2 * N, D)
        acc, den = _edge_kernel(fsfd, eidx, attn[i], zeros)
        den1 = (den[0] + den[1]).reshape(NP)[:N].reshape(N, 1)
        h = _post(acc, den1, h, ln_g[i].reshape(1, D), ln_b[i].reshape(1, D))
    return _mm1(h, W_out, b_out.reshape(1, D))
